# speedup vs baseline: 1.1139x; 1.1139x over previous
import functools

import jax
import jax.numpy as jnp
from jax import lax
from jax.experimental import pallas as pl
from jax.experimental.pallas import tpu as pltpu
from jax.experimental.pallas import tpu_sc as plsc

VOCAB = 100
HIDDEN = 1024
NUM_ROWS = 4096 * 20
NUM_CORES = 2
NUM_SUBCORES = 16
NUM_WORKERS = NUM_CORES * NUM_SUBCORES
ROWS_PER_WORKER = NUM_ROWS // NUM_WORKERS
CHUNK = 40
NBUF = 2
NUM_CHUNKS = ROWS_PER_WORKER // CHUNK

_MESH = plsc.VectorSubcoreMesh(core_axis_name="c", subcore_axis_name="s")


@functools.partial(
    pl.kernel,
    out_type=jax.ShapeDtypeStruct((NUM_ROWS, HIDDEN), jnp.float32),
    mesh=_MESH,
    scratch_types=[
        pltpu.VMEM((ROWS_PER_WORKER,), jnp.int32),
        pltpu.VMEM((NBUF, CHUNK, HIDDEN), jnp.float32),
        pltpu.SemaphoreType.DMA,
        pltpu.SemaphoreType.DMA,
        pltpu.SemaphoreType.DMA,
        pltpu.SemaphoreType.DMA,
    ],
)
def _emb_lookup(idx_hbm, table_hbm, out_hbm, idx_v, bufs, gsem0, gsem1,
                osem0, osem1):
    wid = lax.axis_index("s") * NUM_CORES + lax.axis_index("c")
    base = wid * ROWS_PER_WORKER
    pltpu.sync_copy(idx_hbm.at[pl.ds(base, ROWS_PER_WORKER)], idx_v)

    gsems = (gsem0, gsem1)
    osems = (osem0, osem1)

    def out_slice(g):
        return out_hbm.at[pl.ds(base + g * CHUNK, CHUNK)]

    def body(step, carry):
        g0 = step * NBUF
        for b in range(NBUF):
            g = g0 + b

            @pl.when(g >= NBUF)
            def _():
                pltpu.make_async_copy(
                    bufs.at[b], out_slice(g - NBUF), osems[b]).wait()

            pltpu.async_copy(
                table_hbm.at[idx_v.at[pl.ds(g * CHUNK, CHUNK)]],
                bufs.at[b], gsems[b]).wait()

            pltpu.async_copy(bufs.at[b], out_slice(g), osems[b])
        return carry

    lax.fori_loop(0, NUM_CHUNKS // NBUF, body, 0)

    for b in range(NBUF):
        g = NUM_CHUNKS - NBUF + b
        pltpu.make_async_copy(bufs.at[b], out_slice(g), osems[b]).wait()


def kernel(indices, word_embeddings):
    idx_flat = indices.reshape(NUM_ROWS).astype(jnp.int32)
    out = _emb_lookup(idx_flat, word_embeddings)
    return out.reshape(indices.shape + (HIDDEN,))


# speedup vs baseline: 1.1834x; 1.0624x over previous
import functools

import jax
import jax.numpy as jnp
from jax import lax
from jax.experimental import pallas as pl
from jax.experimental.pallas import tpu as pltpu
from jax.experimental.pallas import tpu_sc as plsc

VOCAB = 100
HIDDEN = 1024
BATCH = 4096
SEQ = 20
SEQ_PAD = 24
NUM_CORES = 2
NUM_SUBCORES = 16
NUM_WORKERS = NUM_CORES * NUM_SUBCORES
BATCH_PER_WORKER = BATCH // NUM_WORKERS
NBUF = 4

_MESH = plsc.VectorSubcoreMesh(core_axis_name="c", subcore_axis_name="s")


@functools.partial(
    pl.kernel,
    out_type=jax.ShapeDtypeStruct((BATCH, SEQ, HIDDEN), jnp.float32),
    mesh=_MESH,
    scratch_types=[
        pltpu.VMEM((BATCH_PER_WORKER * SEQ_PAD,), jnp.int32),
        [pltpu.VMEM((SEQ, HIDDEN), jnp.float32) for _ in range(NBUF)],
        [pltpu.SemaphoreType.DMA] * NBUF,
        [pltpu.SemaphoreType.DMA] * NBUF,
    ],
    compiler_params=pltpu.CompilerParams(use_tc_tiling_on_sc=False),
)
def _emb_lookup(idx_hbm, table_hbm, out_hbm, idx_v, bufs, gsems, osems):
    wid = lax.axis_index("s") * NUM_CORES + lax.axis_index("c")
    bbase = wid * BATCH_PER_WORKER
    pltpu.sync_copy(
        idx_hbm.at[pl.ds(bbase * SEQ_PAD, BATCH_PER_WORKER * SEQ_PAD)], idx_v)

    def body(step, carry):
        k0 = step * NBUF
        for b in range(NBUF):
            k = k0 + b

            @pl.when(k >= NBUF)
            def _():
                pltpu.make_async_copy(
                    bufs[b], out_hbm.at[bbase + k - NBUF], osems[b]).wait()

            pltpu.async_copy(
                table_hbm.at[idx_v.at[pl.ds(k * SEQ_PAD, SEQ)]],
                bufs[b], gsems[b]).wait()

            pltpu.async_copy(bufs[b], out_hbm.at[bbase + k], osems[b])
        return carry

    lax.fori_loop(0, BATCH_PER_WORKER // NBUF, body, 0)

    for b in range(NBUF):
        k = BATCH_PER_WORKER - NBUF + b
        pltpu.make_async_copy(bufs[b], out_hbm.at[bbase + k], osems[b]).wait()


def kernel(indices, word_embeddings):
    idx_pad = jnp.pad(indices.astype(jnp.int32), ((0, 0), (0, SEQ_PAD - SEQ)))
    return _emb_lookup(idx_pad.reshape(BATCH * SEQ_PAD), word_embeddings)


# speedup vs baseline: 1.2032x; 1.0167x over previous
import functools

import jax
import jax.numpy as jnp
from jax import lax
from jax.experimental import pallas as pl
from jax.experimental.pallas import tpu as pltpu
from jax.experimental.pallas import tpu_sc as plsc

VOCAB = 100
HIDDEN = 1024
BATCH = 4096
SEQ = 20
NUM_ROWS = BATCH * SEQ
NUM_CORES = 2
NUM_SUBCORES = 16
NUM_WORKERS = NUM_CORES * NUM_SUBCORES
ROWS_PER_WORKER = NUM_ROWS // NUM_WORKERS
CHUNK = 40
NBUF = 2
NUM_CHUNKS = ROWS_PER_WORKER // CHUNK

_MESH = plsc.VectorSubcoreMesh(core_axis_name="c", subcore_axis_name="s")


@functools.partial(
    pl.kernel,
    out_type=jax.ShapeDtypeStruct((NUM_ROWS, HIDDEN), jnp.float32),
    mesh=_MESH,
    scratch_types=[
        pltpu.VMEM((ROWS_PER_WORKER,), jnp.int32),
        pltpu.VMEM((NBUF, CHUNK, HIDDEN), jnp.float32),
        [pltpu.SemaphoreType.DMA] * NBUF,
        [pltpu.SemaphoreType.DMA] * NBUF,
    ],
)
def _emb_gather(idx_hbm, table_hbm, out_hbm, idx_v, bufs, gsems, osems):
    wid = lax.axis_index("s") * NUM_CORES + lax.axis_index("c")
    base = wid * ROWS_PER_WORKER
    pltpu.sync_copy(idx_hbm.at[pl.ds(base, ROWS_PER_WORKER)], idx_v)

    def out_slice(g):
        return out_hbm.at[pl.ds(base + g * CHUNK, CHUNK)]

    def body(step, carry):
        g0 = step * NBUF
        for b in range(NBUF):
            g = g0 + b

            @pl.when(g >= NBUF)
            def _():
                pltpu.make_async_copy(
                    bufs.at[b], out_slice(g - NBUF), osems[b]).wait()

            pltpu.async_copy(
                table_hbm.at[idx_v.at[pl.ds(g * CHUNK, CHUNK)]],
                bufs.at[b], gsems[b]).wait()

            pltpu.async_copy(bufs.at[b], out_slice(g), osems[b])
        return carry

    lax.fori_loop(0, NUM_CHUNKS // NBUF, body, 0)

    for b in range(NBUF):
        g = NUM_CHUNKS - NBUF + b
        pltpu.make_async_copy(bufs.at[b], out_slice(g), osems[b]).wait()


BB = 64


def _relayout_body(x_ref, o_ref):
    o_ref[...] = x_ref[...].reshape(BB, SEQ, HIDDEN)


_relayout = pl.pallas_call(
    _relayout_body,
    grid=(BATCH // BB,),
    in_specs=[pl.BlockSpec((BB * SEQ, HIDDEN), lambda i: (i, 0))],
    out_specs=pl.BlockSpec((BB, SEQ, HIDDEN), lambda i: (i, 0, 0)),
    out_shape=jax.ShapeDtypeStruct((BATCH, SEQ, HIDDEN), jnp.float32),
)


def kernel(indices, word_embeddings):
    idx_flat = indices.reshape(NUM_ROWS).astype(jnp.int32)
    flat = _emb_gather(idx_flat, word_embeddings)
    return _relayout(flat)


# speedup vs baseline: 2.4147x; 2.0068x over previous
import functools

import jax
import jax.numpy as jnp
from jax import lax
from jax.experimental import pallas as pl
from jax.experimental.pallas import tpu as pltpu
from jax.experimental.pallas import tpu_sc as plsc

VOCAB = 100
HIDDEN = 1024
BATCH = 4096
SEQ = 20
NUM_ROWS = BATCH * SEQ
NUM_CORES = 2
NUM_SUBCORES = 16
NUM_WORKERS = NUM_CORES * NUM_SUBCORES
ROWS_PER_WORKER = NUM_ROWS // NUM_WORKERS
CHUNK = 40
NBUF = 2
NUM_CHUNKS = ROWS_PER_WORKER // CHUNK

_MESH = plsc.VectorSubcoreMesh(core_axis_name="c", subcore_axis_name="s")


@functools.partial(
    pl.kernel,
    out_type=jax.ShapeDtypeStruct((NUM_ROWS, HIDDEN), jnp.float32),
    mesh=_MESH,
    scratch_types=[
        pltpu.VMEM((ROWS_PER_WORKER,), jnp.int32),
        pltpu.VMEM((NBUF, CHUNK, HIDDEN), jnp.float32),
        [pltpu.SemaphoreType.DMA] * NBUF,
        [pltpu.SemaphoreType.DMA] * NBUF,
    ],
)
def _emb_gather(idx_hbm, table_hbm, out_hbm, idx_v, bufs, gsems, osems):
    wid = lax.axis_index("s") * NUM_CORES + lax.axis_index("c")
    base = wid * ROWS_PER_WORKER
    pltpu.sync_copy(idx_hbm.at[pl.ds(base, ROWS_PER_WORKER)], idx_v)

    def out_slice(g):
        return out_hbm.at[pl.ds(base + g * CHUNK, CHUNK)]

    def body(step, carry):
        g0 = step * NBUF
        for b in range(NBUF):
            g = g0 + b

            @pl.when(g >= NBUF)
            def _():
                pltpu.make_async_copy(
                    bufs.at[b], out_slice(g - NBUF), osems[b]).wait()

            pltpu.async_copy(
                table_hbm.at[idx_v.at[pl.ds(g * CHUNK, CHUNK)]],
                bufs.at[b], gsems[b]).wait()

            pltpu.async_copy(bufs.at[b], out_slice(g), osems[b])
        return carry

    lax.fori_loop(0, NUM_CHUNKS // NBUF, body, 0)

    for b in range(NBUF):
        g = NUM_CHUNKS - NBUF + b
        pltpu.make_async_copy(bufs.at[b], out_slice(g), osems[b]).wait()


def kernel(indices, word_embeddings):
    idx_t = indices.T.reshape(NUM_ROWS).astype(jnp.int32)
    flat = _emb_gather(idx_t, word_embeddings)
    return flat.reshape(SEQ, BATCH, HIDDEN).transpose(1, 0, 2)
